# baseline (speedup 1.0000x reference)
import functools

import jax
import jax.numpy as jnp
from jax import lax
from jax.experimental import pallas as pl
from jax.experimental.pallas import tpu as pltpu
from jax.experimental.pallas import tpu_sc as plsc

_K = 128
_D = 32
_L = 8
_B = 4096
_NB = 4096
_G = 8
_NW = 32
_BPW = _B // _NW


def _tc_body(x_ref, cb_ref, cbt_ref, pol_ref, qnt_ref):
    x = x_ref[0]
    cb = cb_ref[...]
    cbt = cbt_ref[...]
    prod = jax.lax.dot_general(
        cb, x, (((1,), (0,)), ((), ())),
        preferred_element_type=jnp.float32)
    dist = (jnp.sum(x * x, axis=0, keepdims=True)
            + jnp.sum(cb * cb, axis=1, keepdims=True)) - 2.0 * prod
    mins = jnp.min(dist, axis=0, keepdims=True)
    iota = jax.lax.broadcasted_iota(jnp.int32, dist.shape, 0)
    idx = jnp.min(jnp.where(dist == mins, iota, _K), axis=0, keepdims=True)
    onehot = (iota == idx).astype(jnp.float32)
    q = jax.lax.dot_general(
        cbt, onehot, (((1,), (0,)), ((), ())),
        preferred_element_type=jnp.float32)
    pol_ref[0] = q
    qnt_ref[0] = q


def _sc_body(cbt_hbm, out_hbm, buf, sem):
    wid = lax.axis_index("s") * 2 + lax.axis_index("c")
    for g in range(_G):
        pltpu.sync_copy(cbt_hbm, buf.at[g])
    base = wid * _BPW
    copies = [
        pltpu.async_copy(buf, out_hbm.at[pl.ds(base + j * _G, _G)], sem)
        for j in range(_BPW // _G)
    ]
    for cp in copies:
        cp.wait()


@functools.partial(jax.jit, static_argnames=())
def kernel(latent, codebook):
    lat_t = latent.transpose(1, 2, 0)
    cbt = codebook.T
    pol, qnt = pl.pallas_call(
        _tc_body,
        grid=(_L,),
        in_specs=[
            pl.BlockSpec((1, _D, _NB), lambda l: (l, 0, 0)),
            pl.BlockSpec((_K, _D), lambda l: (0, 0)),
            pl.BlockSpec((_D, _K), lambda l: (0, 0)),
        ],
        out_specs=[
            pl.BlockSpec((1, _D, _NB), lambda l: (l, 0, 0)),
            pl.BlockSpec((1, _D, _NB), lambda l: (l, 0, 0)),
        ],
        out_shape=[
            jax.ShapeDtypeStruct((_L, _D, _B), jnp.float32),
            jax.ShapeDtypeStruct((_L, _D, _B), jnp.float32),
        ],
        compiler_params=pltpu.CompilerParams(
            dimension_semantics=("arbitrary",),
        ),
    )(lat_t, codebook, cbt)

    sc_bcast = pl.kernel(
        _sc_body,
        out_type=jax.ShapeDtypeStruct((_B, _D, _K), jnp.float32),
        mesh=plsc.VectorSubcoreMesh(core_axis_name="c", subcore_axis_name="s"),
        scratch_types=[
            pltpu.VMEM((_G, _D, _K), jnp.float32),
            pltpu.SemaphoreType.DMA,
        ],
    )
    cset_t = sc_bcast(cbt)

    pol = pol.transpose(2, 0, 1)
    qnt = qnt.transpose(2, 0, 1)
    return (pol, qnt, cset_t.transpose(0, 2, 1))


# speedup vs baseline: 2.1010x; 2.1010x over previous
import functools

import jax
import jax.numpy as jnp
from jax.experimental import pallas as pl
from jax.experimental.pallas import tpu as pltpu

_K = 128
_D = 32
_L = 8
_B = 4096
_NB = 4096
_JG = _B // _NB
_BB = _B // (_L * _JG)


def _body(x_ref, cb_ref, cbt_ref, pol_ref, qnt_ref, cset_ref):
    x = x_ref[0]
    cb = cb_ref[...]
    cbt = cbt_ref[...]
    prod = jax.lax.dot_general(
        cb, x, (((1,), (0,)), ((), ())),
        preferred_element_type=jnp.float32)
    dist = (jnp.sum(x * x, axis=0, keepdims=True)
            + jnp.sum(cb * cb, axis=1, keepdims=True)) - 2.0 * prod
    mins = jnp.min(dist, axis=0, keepdims=True)
    iota = jax.lax.broadcasted_iota(jnp.int32, dist.shape, 0)
    idx = jnp.min(jnp.where(dist == mins, iota, _K), axis=0, keepdims=True)
    onehot = (iota == idx).astype(jnp.float32)
    q = jax.lax.dot_general(
        cbt, onehot, (((1,), (0,)), ((), ())),
        preferred_element_type=jnp.float32)
    pol_ref[0] = q
    qnt_ref[0] = q
    cset_ref[...] = jnp.broadcast_to(cbt[None], cset_ref.shape)


@functools.partial(jax.jit, static_argnames=())
def kernel(latent, codebook):
    lat_t = latent.transpose(1, 2, 0)
    cbt = codebook.T
    pol, qnt, cset_t = pl.pallas_call(
        _body,
        grid=(_L,),
        in_specs=[
            pl.BlockSpec((1, _D, _NB), lambda l: (l, 0, 0)),
            pl.BlockSpec((_K, _D), lambda l: (0, 0)),
            pl.BlockSpec((_D, _K), lambda l: (0, 0)),
        ],
        out_specs=[
            pl.BlockSpec((1, _D, _NB), lambda l: (l, 0, 0)),
            pl.BlockSpec((1, _D, _NB), lambda l: (l, 0, 0)),
            pl.BlockSpec((_BB, _D, _K), lambda l: (l, 0, 0)),
        ],
        out_shape=[
            jax.ShapeDtypeStruct((_L, _D, _B), jnp.float32),
            jax.ShapeDtypeStruct((_L, _D, _B), jnp.float32),
            jax.ShapeDtypeStruct((_B, _D, _K), jnp.float32),
        ],
        compiler_params=pltpu.CompilerParams(
            dimension_semantics=("parallel",),
        ),
    )(lat_t, codebook, cbt)
    pol = pol.transpose(2, 0, 1)
    qnt = qnt.transpose(2, 0, 1)
    return (pol, qnt, cset_t.transpose(0, 2, 1))
